# baseline (speedup 1.0000x reference)
import functools

import jax
import jax.numpy as jnp
from jax import lax
from jax.experimental import pallas as pl
from jax.experimental.pallas import tpu as pltpu
from jax.experimental.pallas import tpu_sc as plsc

_NC = 2
_NS = 16
_NW = _NC * _NS
_NB = 8
_SP = 56
_DP = 128


@functools.lru_cache(maxsize=None)
def _make_kernel(R, S, SI):
    rows_per_w = R // _NW
    n_chunks = rows_per_w // _NB
    mesh = plsc.VectorSubcoreMesh(core_axis_name="c", subcore_axis_name="s")

    @functools.partial(
        pl.kernel,
        mesh=mesh,
        compiler_params=pltpu.CompilerParams(use_tc_tiling_on_sc=False),
        out_type=jax.ShapeDtypeStruct((R * _SP, _DP), jnp.float32),
        scratch_types=[
            pltpu.VMEM((_NB, SI), jnp.int32),
            pltpu.VMEM((_NB, SI), jnp.int32),
            pltpu.VMEM((_NB * _SP, _DP), jnp.float32),
            pltpu.VMEM((_NB * _SP, _DP), jnp.float32),
            pltpu.SemaphoreType.DMA,
            pltpu.SemaphoreType.DMA,
            pltpu.SemaphoreType.DMA,
            pltpu.SemaphoreType.DMA,
        ],
    )
    def k(ind_hbm, w_hbm, out_hbm, idx0, idx1, rows0, rows1, gs0, gs1,
          ws0, ws1):
        wid = lax.axis_index("s") * _NC + lax.axis_index("c")
        row0 = wid * rows_per_w
        idxs = (idx0, idx1)
        rows = (rows0, rows1)
        gsems = (gs0, gs1)
        wsems = (ws0, ws1)

        def wait_gathers(b):
            for j in range(_NB):
                pltpu.make_async_copy(
                    w_hbm.at[idxs[b].at[j, pl.ds(0, _SP)]],
                    rows[b].at[pl.ds(j * _SP, _SP)], gsems[b]).wait()

        def wait_writeback(b):
            pltpu.make_async_copy(
                rows[b], out_hbm.at[pl.ds(row0 * _SP, _NB * _SP)],
                wsems[b]).wait()

        def do_chunk(c, b):
            pltpu.sync_copy(ind_hbm.at[pl.ds(row0 + c * _NB, _NB), :],
                            idxs[b])
            for j in range(_NB):
                pltpu.async_copy(w_hbm.at[idxs[b].at[j, pl.ds(0, _SP)]],
                                 rows[b].at[pl.ds(j * _SP, _SP)], gsems[b])

        def writeback(c, b):
            pltpu.async_copy(
                rows[b], out_hbm.at[pl.ds((row0 + c * _NB) * _SP, _NB * _SP)],
                wsems[b])

        @pl.loop(0, n_chunks, step=2)
        def chunk_loop(iv):
            @pl.when(iv >= 2)
            def _():
                wait_writeback(0)
            do_chunk(iv, 0)

            @pl.when(iv >= 2)
            def _():
                wait_gathers(1)
                writeback(iv - 1, 1)

            @pl.when(iv >= 2)
            def _():
                wait_writeback(1)
            do_chunk(iv + 1, 1)
            wait_gathers(0)
            writeback(iv, 0)

        wait_gathers(1)
        writeback(n_chunks - 1, 1)
        wait_writeback(0)
        wait_writeback(1)

    return k


def kernel(ind, weight):
    R, S = ind.shape
    V, D = weight.shape
    ind_p = jnp.pad(ind.astype(jnp.int32), ((0, 0), (0, 64 - S)))
    w_p = jnp.pad(weight, ((0, 0), (0, _DP - D)))
    out2d = _make_kernel(R, S, 64)(ind_p, w_p)
    return out2d.reshape(R, _SP, _DP)[:, :S, :D]


# speedup vs baseline: 1.6289x; 1.6289x over previous
import functools

import jax
import jax.numpy as jnp
from jax import lax
from jax.experimental import pallas as pl
from jax.experimental.pallas import tpu as pltpu
from jax.experimental.pallas import tpu_sc as plsc

_NC = 2
_NS = 16
_NW = _NC * _NS
_NB = 8
_SP = 56


@functools.lru_cache(maxsize=None)
def _make_kernel(R, S, D):
    rows_per_w = R // _NW
    n_chunks = rows_per_w // _NB
    ni = _NB * _SP
    mesh = plsc.VectorSubcoreMesh(core_axis_name="c", subcore_axis_name="s")

    @functools.partial(
        pl.kernel,
        mesh=mesh,
        compiler_params=pltpu.CompilerParams(use_tc_tiling_on_sc=False),
        out_type=jax.ShapeDtypeStruct((R, S, D), jnp.float32),
        scratch_types=[
            pltpu.VMEM((ni,), jnp.int32),
            pltpu.VMEM((ni,), jnp.int32),
            pltpu.VMEM((ni, D), jnp.float32),
            pltpu.VMEM((ni, D), jnp.float32),
            pltpu.SemaphoreType.DMA,
            pltpu.SemaphoreType.DMA,
            pltpu.SemaphoreType.DMA,
            pltpu.SemaphoreType.DMA,
        ],
    )
    def k(ind_hbm, w_hbm, out_hbm, idx0, idx1, rows0, rows1, gs0, gs1,
          ws0, ws1):
        wid = lax.axis_index("s") * _NC + lax.axis_index("c")
        row0 = wid * rows_per_w
        idxs = (idx0, idx1)
        rows = (rows0, rows1)
        gsems = (gs0, gs1)
        wsems = (ws0, ws1)

        def wait_gather(b):
            pltpu.make_async_copy(w_hbm.at[idxs[b]], rows[b],
                                  gsems[b]).wait()

        def wait_writeback(b):
            for j in range(_NB):
                pltpu.make_async_copy(
                    rows[b].at[pl.ds(j * _SP, S)], out_hbm.at[row0 + j],
                    wsems[b]).wait()

        def do_chunk(c, b):
            pltpu.sync_copy(
                ind_hbm.at[pl.ds((row0 + c * _NB) * _SP, ni)], idxs[b])
            pltpu.async_copy(w_hbm.at[idxs[b]], rows[b], gsems[b])

        def writeback(c, b):
            for j in range(_NB):
                pltpu.async_copy(
                    rows[b].at[pl.ds(j * _SP, S)],
                    out_hbm.at[row0 + c * _NB + j], wsems[b])

        @pl.loop(0, n_chunks, step=2)
        def chunk_loop(iv):
            @pl.when(iv >= 2)
            def _():
                wait_writeback(0)
            do_chunk(iv, 0)

            @pl.when(iv >= 2)
            def _():
                wait_gather(1)
                writeback(iv - 1, 1)

            @pl.when(iv >= 2)
            def _():
                wait_writeback(1)
            do_chunk(iv + 1, 1)
            wait_gather(0)
            writeback(iv, 0)

        wait_gather(1)
        writeback(n_chunks - 1, 1)
        wait_writeback(0)
        wait_writeback(1)

    return k


def kernel(ind, weight):
    R, S = ind.shape
    V, D = weight.shape
    ind_p = jnp.pad(ind.astype(jnp.int32), ((0, 0), (0, _SP - S)))
    return _make_kernel(R, S, D)(ind_p.reshape(-1), weight)


# speedup vs baseline: 5.2224x; 3.2061x over previous
import functools

import jax
import jax.numpy as jnp
from jax import lax
from jax.experimental import pallas as pl
from jax.experimental.pallas import tpu as pltpu
from jax.experimental.pallas import tpu_sc as plsc

_NC = 2
_NS = 16
_NW = _NC * _NS


@functools.lru_cache(maxsize=None)
def _make_kernel(B, V, D, chunk):
    b_per_w = B // _NW
    n_chunks = b_per_w // chunk
    mesh = plsc.VectorSubcoreMesh(core_axis_name="c", subcore_axis_name="s")

    @functools.partial(
        pl.kernel,
        mesh=mesh,
        compiler_params=pltpu.CompilerParams(use_tc_tiling_on_sc=False),
        out_type=jax.ShapeDtypeStruct((B, D), jnp.float32),
        scratch_types=[
            pltpu.VMEM((b_per_w,), jnp.int32),
            pltpu.VMEM((chunk, D), jnp.float32),
            pltpu.VMEM((chunk, D), jnp.float32),
            pltpu.SemaphoreType.DMA,
            pltpu.SemaphoreType.DMA,
            pltpu.SemaphoreType.DMA,
            pltpu.SemaphoreType.DMA,
        ],
    )
    def k(idx_hbm, table_hbm, out_hbm, idx_v, buf0, buf1, gs0, gs1, ws0, ws1):
        wid = lax.axis_index("s") * _NC + lax.axis_index("c")
        base = wid * b_per_w
        pltpu.sync_copy(idx_hbm.at[pl.ds(base, b_per_w)], idx_v)
        bufs = (buf0, buf1)
        gsems = (gs0, gs1)
        wsems = (ws0, ws1)

        def gather(c):
            return pltpu.async_copy(
                table_hbm.at[idx_v.at[pl.ds(c * chunk, chunk)]],
                bufs[c % 2], gsems[c % 2])

        gh = {0: gather(0)}
        if n_chunks > 1:
            gh[1] = gather(1)
        wh = {}
        for c in range(n_chunks):
            gh[c].wait()
            wh[c] = pltpu.async_copy(
                bufs[c % 2], out_hbm.at[pl.ds(base + c * chunk, chunk)],
                wsems[c % 2])
            if c + 2 < n_chunks:
                wh[c].wait()
                gh[c + 2] = gather(c + 2)
        for c in range(max(0, n_chunks - 2), n_chunks):
            wh[c].wait()

    return k


def kernel(ind, weight):
    ind_shape = ind.shape
    flat = ind.reshape(-1).astype(jnp.int32)
    B = flat.shape[0]
    V, D = weight.shape
    out = _make_kernel(B, V, D, 800)(flat, weight)
    return out.reshape(*ind_shape, D)
